# Initial kernel scaffold; baseline (speedup 1.0000x reference)
import jax
import jax.numpy as jnp
from jax.experimental import pallas as pl


def kernel(x, W_enc, b_enc, W_dec, b_dec):
    raise NotImplementedError("write your pallas kernel here")


# speedup vs baseline: 19.2654x; 19.2654x over previous
import functools

import jax
import jax.numpy as jnp
from jax import lax
from jax.experimental import pallas as pl
from jax.experimental.pallas import tpu as pltpu
from jax.experimental.pallas import tpu_sc as plsc

ACT = 1024
DICT = 16384
BATCH = 4096
K_TOTAL = 64 * BATCH

N = BATCH * DICT
NW = 32
PER_W = N // NW
CHUNK = 8192
NCH = PER_W // CHUNK
NB12 = 4096
NB8 = 256
UNROLL = 8

_mesh = plsc.VectorSubcoreMesh(core_axis_name="c", subcore_axis_name="s")
_sc_params = pltpu.CompilerParams(needs_layout_passes=False)


def _wid():
    return lax.axis_index("s") * 2 + lax.axis_index("c")


def _iota16():
    return lax.iota(jnp.int32, 16)


def _zero_ref(ref, n):
    def body(i, _):
        ref[pl.ds(i * 16, 16)] = jnp.zeros((16,), jnp.int32)
        return 0
    lax.fori_loop(0, n // 16, body, 0)


def _scalar(x):
    return x if x.ndim == 0 else x[0]


def _merge_hist(h_hbm, acc_ref, row_ref, nb):
    nch = nb // 16
    _zero_ref(acc_ref, nb)

    def rowbody(rid, _):
        pltpu.sync_copy(h_hbm.at[rid], row_ref.at[pl.ds(0, nb)])

        def add(i, _):
            acc_ref[pl.ds(i * 16, 16)] = (
                acc_ref[pl.ds(i * 16, 16)] + row_ref[pl.ds(i * 16, 16)]
            )
            return 0

        lax.fori_loop(0, nch, add, 0)
        return 0

    lax.fori_loop(0, NW, rowbody, 0)


def _find_crossing(acc_ref, nb, r):
    nchunks = nb // 16
    lanes = _iota16()

    def body(i, carry):
        accum, found, bstar, rnext = carry
        c = nchunks - 1 - i
        v = acc_ref[pl.ds(c * 16, 16)]
        w = lax.rev(v, (0,))
        cw = jnp.cumsum(w)
        m = (accum + cw) >= r
        pc = _scalar(plsc.all_reduce_population_count(m))
        has = jnp.logical_and(jnp.logical_not(found), pc > 0)
        j = 16 - pc
        cwj = jnp.sum(jnp.where(lanes == j, cw, 0))
        wj = jnp.sum(jnp.where(lanes == j, w, 0))
        bstar = jnp.where(has, c * 16 + (15 - j), bstar)
        rnext = jnp.where(has, r - (accum + cwj - wj), rnext)
        found = jnp.logical_or(found, pc > 0)
        accum = accum + jnp.sum(v)
        return accum, found, bstar, rnext

    _, _, bstar, rnext = lax.fori_loop(
        0, nchunks, body,
        (jnp.int32(0), jnp.bool_(False), jnp.int32(0), jnp.int32(0)),
    )
    return bstar, rnext


def _scan_chunks(mu_hbm, hist_ref, buf_ref, wid, level, nb, b1, b2):
    lane_base = _iota16() * nb
    ones = jnp.ones((16,), jnp.int32)
    base_w = wid * PER_W
    prefix24 = None if level < 3 else ((b1 << 12) | b2)

    def chunk_body(c, _):
        pltpu.sync_copy(mu_hbm.at[pl.ds(base_w + c * CHUNK, CHUNK)], buf_ref)

        def inner(i, _):
            for u in range(UNROLL):
                v = buf_ref[pl.ds((i * UNROLL + u) * 16, 16)]
                bits = lax.bitcast_convert_type(v, jnp.int32)
                bpos = jnp.maximum(bits, 0)
                if level == 1:
                    idx = lane_base + (bpos >> 20)
                    plsc.addupdate_scatter(hist_ref, [idx], ones)
                elif level == 2:
                    idx = lane_base + ((bpos >> 8) & 0xFFF)
                    plsc.addupdate_scatter(
                        hist_ref, [idx], ones, mask=(bpos >> 20) == b1)
                else:
                    idx = lane_base + (bpos & 0xFF)
                    plsc.addupdate_scatter(
                        hist_ref, [idx], ones, mask=(bpos >> 8) == prefix24)
            return 0

        lax.fori_loop(0, CHUNK // (16 * UNROLL), inner, 0)
        return 0

    lax.fori_loop(0, NCH, chunk_body, 0)


def _merge_lanes_and_store(hist_ref, acc_ref, out_hbm, wid, nb):
    def body(i, _):
        s = hist_ref[pl.ds(i * 16, 16)]
        for ln in range(1, 16):
            s = s + hist_ref[pl.ds(ln * nb + i * 16, 16)]
        acc_ref[pl.ds(i * 16, 16)] = s
        return 0

    lax.fori_loop(0, nb // 16, body, 0)
    pltpu.sync_copy(acc_ref.at[pl.ds(0, nb)], out_hbm.at[wid])


@functools.partial(
    pl.kernel, mesh=_mesh, compiler_params=_sc_params,
    out_type=jax.ShapeDtypeStruct((NW, NB12), jnp.int32),
    scratch_types=[
        pltpu.VMEM((16 * NB12,), jnp.int32),
        pltpu.VMEM((NB12,), jnp.int32),
        pltpu.VMEM((CHUNK,), jnp.float32),
    ],
)
def _sc_level1(mu_hbm, h1_out, hist_v, acc_v, buf_v):
    wid = _wid()
    _zero_ref(hist_v, 16 * NB12)
    _scan_chunks(mu_hbm, hist_v, buf_v, wid, 1, NB12, None, None)
    _merge_lanes_and_store(hist_v, acc_v, h1_out, wid, NB12)


@functools.partial(
    pl.kernel, mesh=_mesh, compiler_params=_sc_params,
    out_type=jax.ShapeDtypeStruct((NW, NB12), jnp.int32),
    scratch_types=[
        pltpu.VMEM((16 * NB12,), jnp.int32),
        pltpu.VMEM((NB12,), jnp.int32),
        pltpu.VMEM((NB12,), jnp.int32),
        pltpu.VMEM((CHUNK,), jnp.float32),
    ],
)
def _sc_level2(mu_hbm, h1_hbm, h2_out, hist_v, acc_v, row_v, buf_v):
    wid = _wid()
    _merge_hist(h1_hbm, acc_v, row_v, NB12)
    b1, _ = _find_crossing(acc_v, NB12, K_TOTAL)
    _zero_ref(hist_v, 16 * NB12)
    _scan_chunks(mu_hbm, hist_v, buf_v, wid, 2, NB12, b1, None)
    _merge_lanes_and_store(hist_v, acc_v, h2_out, wid, NB12)


@functools.partial(
    pl.kernel, mesh=_mesh, compiler_params=_sc_params,
    out_type=jax.ShapeDtypeStruct((NW, NB8), jnp.int32),
    scratch_types=[
        pltpu.VMEM((16 * NB8,), jnp.int32),
        pltpu.VMEM((NB12,), jnp.int32),
        pltpu.VMEM((NB12,), jnp.int32),
        pltpu.VMEM((CHUNK,), jnp.float32),
    ],
)
def _sc_level3(mu_hbm, h1_hbm, h2_hbm, h3_out, hist_v, acc_v, row_v, buf_v):
    wid = _wid()
    _merge_hist(h1_hbm, acc_v, row_v, NB12)
    b1, r1 = _find_crossing(acc_v, NB12, K_TOTAL)
    _merge_hist(h2_hbm, acc_v, row_v, NB12)
    b2, _ = _find_crossing(acc_v, NB12, r1)
    _zero_ref(hist_v, 16 * NB8)
    _scan_chunks(mu_hbm, hist_v, buf_v, wid, 3, NB8, b1, b2)
    _merge_lanes_and_store(hist_v, acc_v, h3_out, wid, NB8)


@functools.partial(
    pl.kernel, mesh=_mesh, compiler_params=_sc_params,
    out_type=jax.ShapeDtypeStruct((16,), jnp.int32),
    scratch_types=[
        pltpu.VMEM((NB12,), jnp.int32),
        pltpu.VMEM((NB12,), jnp.int32),
        pltpu.VMEM((16,), jnp.int32),
    ],
)
def _sc_finalize(h1_hbm, h2_hbm, h3_hbm, tau_out, acc_v, row_v, stage_v):
    wid = _wid()

    @pl.when(wid == 0)
    def _():
        _merge_hist(h1_hbm, acc_v, row_v, NB12)
        b1, r1 = _find_crossing(acc_v, NB12, K_TOTAL)
        _merge_hist(h2_hbm, acc_v, row_v, NB12)
        b2, r2 = _find_crossing(acc_v, NB12, r1)
        _merge_hist(h3_hbm, acc_v, row_v, NB8)
        b3, _ = _find_crossing(acc_v, NB8, r2)
        tau_bits = (b1 << 20) | (b2 << 8) | b3
        stage_v[...] = jnp.broadcast_to(tau_bits, (16,))
        pltpu.sync_copy(stage_v, tau_out)


BM_E = 1024
BN_E = 2048


def _enc_block(x_ref, w_ref, b_ref, o_ref):
    acc = lax.dot_general(
        x_ref[...], w_ref[...], (((1,), (1,)), ((), ())),
        preferred_element_type=jnp.float32,
        precision=lax.Precision.DEFAULT,
    )
    o_ref[...] = jnp.maximum(acc + b_ref[...], 0.0)


def _encoder(x, w_enc, b_enc):
    return pl.pallas_call(
        _enc_block,
        grid=(BATCH // BM_E, DICT // BN_E),
        in_specs=[
            pl.BlockSpec((BM_E, ACT), lambda i, j: (i, 0)),
            pl.BlockSpec((BN_E, ACT), lambda i, j: (j, 0)),
            pl.BlockSpec((1, BN_E), lambda i, j: (0, j)),
        ],
        out_specs=pl.BlockSpec((BM_E, BN_E), lambda i, j: (i, j)),
        out_shape=jax.ShapeDtypeStruct((BATCH, DICT), jnp.float32),
        compiler_params=pltpu.CompilerParams(
            dimension_semantics=("parallel", "parallel")),
    )(x, w_enc, b_enc.reshape(1, DICT))


BM_D = 1024
BK_D = 1024


def _dec_block(mu_ref, w_ref, b_ref, tau_ref, o_ref):
    k = pl.program_id(1)
    tau = tau_ref[0]
    m = mu_ref[...]
    m = jnp.where(m >= tau, m, 0.0)
    acc = lax.dot_general(
        m, w_ref[...], (((1,), (1,)), ((), ())),
        preferred_element_type=jnp.float32,
        precision=lax.Precision.DEFAULT,
    )

    @pl.when(k == 0)
    def _():
        o_ref[...] = acc + b_ref[...]

    @pl.when(k > 0)
    def _():
        o_ref[...] += acc


def _decoder(mu, w_dec, b_dec, tau):
    return pl.pallas_call(
        _dec_block,
        grid=(BATCH // BM_D, DICT // BK_D),
        in_specs=[
            pl.BlockSpec((BM_D, BK_D), lambda i, k: (i, k)),
            pl.BlockSpec((ACT, BK_D), lambda i, k: (0, k)),
            pl.BlockSpec((1, ACT), lambda i, k: (0, 0)),
            pl.BlockSpec(memory_space=pltpu.SMEM),
        ],
        out_specs=pl.BlockSpec((BM_D, ACT), lambda i, k: (i, 0)),
        out_shape=jax.ShapeDtypeStruct((BATCH, ACT), jnp.float32),
        compiler_params=pltpu.CompilerParams(
            dimension_semantics=("parallel", "arbitrary")),
    )(mu, w_dec, b_dec.reshape(1, ACT), tau)


def kernel(x, W_enc, b_enc, W_dec, b_dec):
    mu = _encoder(x, W_enc, b_enc)
    muf = mu.reshape(N)
    h1 = _sc_level1(muf)
    h2 = _sc_level2(muf, h1)
    h3 = _sc_level3(muf, h1, h2)
    taub = _sc_finalize(h1, h2, h3)
    tau = lax.bitcast_convert_type(taub[:1], jnp.float32)
    return _decoder(mu, W_dec, b_dec, tau)


# speedup vs baseline: 58.0214x; 3.0117x over previous
import functools

import jax
import jax.numpy as jnp
from jax import lax
from jax.experimental import pallas as pl
from jax.experimental.pallas import tpu as pltpu
from jax.experimental.pallas import tpu_sc as plsc

ACT = 1024
DICT = 16384
BATCH = 4096
K_TOTAL = 64 * BATCH

N = BATCH * DICT
NW = 32
PER_W = N // NW
CHUNK = 16384
NCH = PER_W // CHUNK
NB12 = 4096
NB8 = 256
UNROLL = 8

_mesh = plsc.VectorSubcoreMesh(core_axis_name="c", subcore_axis_name="s")
_sc_params = pltpu.CompilerParams(needs_layout_passes=False)


def _wid():
    return lax.axis_index("s") * 2 + lax.axis_index("c")


def _iota16():
    return lax.iota(jnp.int32, 16)


def _zero_ref(ref, n):
    @plsc.parallel_loop(0, n // 16, unroll=8)
    def _(i):
        ref[pl.ds(i * 16, 16)] = jnp.zeros((16,), jnp.int32)


def _scalar(x):
    return x if x.ndim == 0 else x[0]


def _merge_hist(h_hbm, acc_ref, row_ref, nb):
    nch = nb // 16
    _zero_ref(acc_ref, nb)

    def rowbody(rid, _):
        pltpu.sync_copy(h_hbm.at[rid], row_ref.at[pl.ds(0, nb)])

        @plsc.parallel_loop(0, nch, unroll=8)
        def _(i):
            acc_ref[pl.ds(i * 16, 16)] = (
                acc_ref[pl.ds(i * 16, 16)] + row_ref[pl.ds(i * 16, 16)]
            )
        return 0

    lax.fori_loop(0, NW, rowbody, 0)


def _find_crossing(acc_ref, nb, r):
    nchunks = nb // 16
    lanes = _iota16()

    def body(i, carry):
        accum, found, bstar, rnext = carry
        c = nchunks - 1 - i
        v = acc_ref[pl.ds(c * 16, 16)]
        w = lax.rev(v, (0,))
        cw = jnp.cumsum(w)
        m = (accum + cw) >= r
        pc = _scalar(plsc.all_reduce_population_count(m))
        has = jnp.logical_and(jnp.logical_not(found), pc > 0)
        j = 16 - pc
        cwj = jnp.sum(jnp.where(lanes == j, cw, 0))
        wj = jnp.sum(jnp.where(lanes == j, w, 0))
        bstar = jnp.where(has, c * 16 + (15 - j), bstar)
        rnext = jnp.where(has, r - (accum + cwj - wj), rnext)
        found = jnp.logical_or(found, pc > 0)
        accum = accum + jnp.sum(v)
        return accum, found, bstar, rnext

    _, _, bstar, rnext = lax.fori_loop(
        0, nchunks, body,
        (jnp.int32(0), jnp.bool_(False), jnp.int32(0), jnp.int32(0)),
    )
    return bstar, rnext


def _scan_chunks(mu_hbm, hist_ref, buf0, buf1, sem0, sem1,
                 wid, level, nb, b1, b2):
    lane_base = _iota16() * nb
    ones = jnp.ones((16,), jnp.int32)
    base_w = wid * PER_W
    prefix24 = None if level < 3 else ((b1 << 12) | b2)

    def _slice(c):
        return mu_hbm.at[pl.ds(base_w + c * CHUNK, CHUNK)]

    def process(buf):
        @plsc.parallel_loop(0, CHUNK // 16, unroll=8)
        def _(i):
            v = buf[pl.ds(i * 16, 16)]
            bits = lax.bitcast_convert_type(v, jnp.int32)
            bpos = jnp.maximum(bits, 0)
            if level == 1:
                idx = lane_base + (bpos >> 20)
                plsc.addupdate_scatter(hist_ref, [idx], ones)
            elif level == 2:
                idx = lane_base + ((bpos >> 8) & 0xFFF)
                plsc.addupdate_scatter(
                    hist_ref, [idx], ones, mask=(bpos >> 20) == b1)
            else:
                idx = lane_base + (bpos & 0xFF)
                plsc.addupdate_scatter(
                    hist_ref, [idx], ones, mask=(bpos >> 8) == prefix24)

    pltpu.async_copy(_slice(0), buf0, sem0)

    def pair(p, _):
        c = 2 * p
        pltpu.async_copy(_slice(c + 1), buf1, sem1)
        pltpu.make_async_copy(_slice(c), buf0, sem0).wait()
        process(buf0)

        @pl.when(c + 2 < NCH)
        def _():
            pltpu.async_copy(_slice(c + 2), buf0, sem0)

        pltpu.make_async_copy(_slice(c + 1), buf1, sem1).wait()
        process(buf1)
        return 0

    lax.fori_loop(0, NCH // 2, pair, 0)


def _merge_lanes_and_store(hist_ref, acc_ref, out_hbm, wid, nb):
    @plsc.parallel_loop(0, nb // 16, unroll=4)
    def _(i):
        s = hist_ref[pl.ds(i * 16, 16)]
        for ln in range(1, 16):
            s = s + hist_ref[pl.ds(ln * nb + i * 16, 16)]
        acc_ref[pl.ds(i * 16, 16)] = s
    pltpu.sync_copy(acc_ref.at[pl.ds(0, nb)], out_hbm.at[wid])


@functools.partial(
    pl.kernel, mesh=_mesh, compiler_params=_sc_params,
    out_type=jax.ShapeDtypeStruct((NW, NB12), jnp.int32),
    scratch_types=[
        pltpu.VMEM((16 * NB12,), jnp.int32),
        pltpu.VMEM((NB12,), jnp.int32),
        pltpu.VMEM((CHUNK,), jnp.float32),
        pltpu.VMEM((CHUNK,), jnp.float32),
        pltpu.SemaphoreType.DMA,
        pltpu.SemaphoreType.DMA,
    ],
)
def _sc_level1(mu_hbm, h1_out, hist_v, acc_v, buf0_v, buf1_v, sem0, sem1):
    wid = _wid()
    _zero_ref(hist_v, 16 * NB12)
    _scan_chunks(mu_hbm, hist_v, buf0_v, buf1_v, sem0, sem1,
                 wid, 1, NB12, None, None)
    _merge_lanes_and_store(hist_v, acc_v, h1_out, wid, NB12)


@functools.partial(
    pl.kernel, mesh=_mesh, compiler_params=_sc_params,
    out_type=jax.ShapeDtypeStruct((NW, NB12), jnp.int32),
    scratch_types=[
        pltpu.VMEM((16 * NB12,), jnp.int32),
        pltpu.VMEM((NB12,), jnp.int32),
        pltpu.VMEM((NB12,), jnp.int32),
        pltpu.VMEM((CHUNK,), jnp.float32),
        pltpu.VMEM((CHUNK,), jnp.float32),
        pltpu.SemaphoreType.DMA,
        pltpu.SemaphoreType.DMA,
    ],
)
def _sc_level2(mu_hbm, h1_hbm, h2_out, hist_v, acc_v, row_v,
               buf0_v, buf1_v, sem0, sem1):
    wid = _wid()
    _merge_hist(h1_hbm, acc_v, row_v, NB12)
    b1, _ = _find_crossing(acc_v, NB12, K_TOTAL)
    _zero_ref(hist_v, 16 * NB12)
    _scan_chunks(mu_hbm, hist_v, buf0_v, buf1_v, sem0, sem1,
                 wid, 2, NB12, b1, None)
    _merge_lanes_and_store(hist_v, acc_v, h2_out, wid, NB12)


@functools.partial(
    pl.kernel, mesh=_mesh, compiler_params=_sc_params,
    out_type=jax.ShapeDtypeStruct((NW, NB8), jnp.int32),
    scratch_types=[
        pltpu.VMEM((16 * NB8,), jnp.int32),
        pltpu.VMEM((NB12,), jnp.int32),
        pltpu.VMEM((NB12,), jnp.int32),
        pltpu.VMEM((CHUNK,), jnp.float32),
        pltpu.VMEM((CHUNK,), jnp.float32),
        pltpu.SemaphoreType.DMA,
        pltpu.SemaphoreType.DMA,
    ],
)
def _sc_level3(mu_hbm, h1_hbm, h2_hbm, h3_out, hist_v, acc_v, row_v,
               buf0_v, buf1_v, sem0, sem1):
    wid = _wid()
    _merge_hist(h1_hbm, acc_v, row_v, NB12)
    b1, r1 = _find_crossing(acc_v, NB12, K_TOTAL)
    _merge_hist(h2_hbm, acc_v, row_v, NB12)
    b2, _ = _find_crossing(acc_v, NB12, r1)
    _zero_ref(hist_v, 16 * NB8)
    _scan_chunks(mu_hbm, hist_v, buf0_v, buf1_v, sem0, sem1,
                 wid, 3, NB8, b1, b2)
    _merge_lanes_and_store(hist_v, acc_v, h3_out, wid, NB8)


@functools.partial(
    pl.kernel, mesh=_mesh, compiler_params=_sc_params,
    out_type=jax.ShapeDtypeStruct((16,), jnp.int32),
    scratch_types=[
        pltpu.VMEM((NB12,), jnp.int32),
        pltpu.VMEM((NB12,), jnp.int32),
        pltpu.VMEM((16,), jnp.int32),
    ],
)
def _sc_finalize(h1_hbm, h2_hbm, h3_hbm, tau_out, acc_v, row_v, stage_v):
    wid = _wid()

    @pl.when(wid == 0)
    def _():
        _merge_hist(h1_hbm, acc_v, row_v, NB12)
        b1, r1 = _find_crossing(acc_v, NB12, K_TOTAL)
        _merge_hist(h2_hbm, acc_v, row_v, NB12)
        b2, r2 = _find_crossing(acc_v, NB12, r1)
        _merge_hist(h3_hbm, acc_v, row_v, NB8)
        b3, _ = _find_crossing(acc_v, NB8, r2)
        tau_bits = (b1 << 20) | (b2 << 8) | b3
        stage_v[...] = jnp.broadcast_to(tau_bits, (16,))
        pltpu.sync_copy(stage_v, tau_out)


BM_E = 1024
BN_E = 2048


def _enc_block(x_ref, w_ref, b_ref, o_ref):
    acc = lax.dot_general(
        x_ref[...], w_ref[...], (((1,), (1,)), ((), ())),
        preferred_element_type=jnp.float32,
        precision=lax.Precision.DEFAULT,
    )
    o_ref[...] = jnp.maximum(acc + b_ref[...], 0.0)


def _encoder(x, w_enc, b_enc):
    return pl.pallas_call(
        _enc_block,
        grid=(BATCH // BM_E, DICT // BN_E),
        in_specs=[
            pl.BlockSpec((BM_E, ACT), lambda i, j: (i, 0)),
            pl.BlockSpec((BN_E, ACT), lambda i, j: (j, 0)),
            pl.BlockSpec((1, BN_E), lambda i, j: (0, j)),
        ],
        out_specs=pl.BlockSpec((BM_E, BN_E), lambda i, j: (i, j)),
        out_shape=jax.ShapeDtypeStruct((BATCH, DICT), jnp.float32),
        compiler_params=pltpu.CompilerParams(
            dimension_semantics=("parallel", "parallel")),
    )(x, w_enc, b_enc.reshape(1, DICT))


BM_D = 1024
BK_D = 1024


def _dec_block(mu_ref, w_ref, b_ref, tau_ref, o_ref):
    k = pl.program_id(1)
    tau = tau_ref[0]
    m = mu_ref[...]
    m = jnp.where(m >= tau, m, 0.0)
    acc = lax.dot_general(
        m, w_ref[...], (((1,), (1,)), ((), ())),
        preferred_element_type=jnp.float32,
        precision=lax.Precision.DEFAULT,
    )

    @pl.when(k == 0)
    def _():
        o_ref[...] = acc + b_ref[...]

    @pl.when(k > 0)
    def _():
        o_ref[...] += acc


def _decoder(mu, w_dec, b_dec, tau):
    return pl.pallas_call(
        _dec_block,
        grid=(BATCH // BM_D, DICT // BK_D),
        in_specs=[
            pl.BlockSpec((BM_D, BK_D), lambda i, k: (i, k)),
            pl.BlockSpec((ACT, BK_D), lambda i, k: (0, k)),
            pl.BlockSpec((1, ACT), lambda i, k: (0, 0)),
            pl.BlockSpec(memory_space=pltpu.SMEM),
        ],
        out_specs=pl.BlockSpec((BM_D, ACT), lambda i, k: (i, 0)),
        out_shape=jax.ShapeDtypeStruct((BATCH, ACT), jnp.float32),
        compiler_params=pltpu.CompilerParams(
            dimension_semantics=("parallel", "arbitrary")),
    )(mu, w_dec, b_dec.reshape(1, ACT), tau)


def kernel(x, W_enc, b_enc, W_dec, b_dec):
    mu = _encoder(x, W_enc, b_enc)
    muf = mu.reshape(N)
    h1 = _sc_level1(muf)
    h2 = _sc_level2(muf, h1)
    h3 = _sc_level3(muf, h1, h2)
    taub = _sc_finalize(h1, h2, h3)
    tau = lax.bitcast_convert_type(taub[:1], jnp.float32)
    return _decoder(mu, W_dec, b_dec, tau)
